# Initial kernel scaffold; baseline (speedup 1.0000x reference)
import jax
import jax.numpy as jnp
from jax.experimental import pallas as pl


def kernel(x, emb):
    raise NotImplementedError("write your pallas kernel here")


# speedup vs baseline: 3.0182x; 3.0182x over previous
import jax
import jax.numpy as jnp
from jax.experimental import pallas as pl

_DIM = 1024
_SCALE = _DIM ** (-0.5)


def _scale_body(e_ref, o_ref):
    o_ref[...] = e_ref[...] * _SCALE


def kernel(x, emb):
    seq_len = x.shape[1]
    rows_per_block = 1024
    grid = (seq_len // rows_per_block,)
    return pl.pallas_call(
        _scale_body,
        grid=grid,
        in_specs=[pl.BlockSpec((rows_per_block, _DIM), lambda i: (i, 0))],
        out_specs=pl.BlockSpec((rows_per_block, _DIM), lambda i: (i, 0)),
        out_shape=jax.ShapeDtypeStruct((seq_len, _DIM), emb.dtype),
    )(emb[:seq_len])
